# baseline (speedup 1.0000x reference)
import functools

import jax
import jax.numpy as jnp
from jax import lax
from jax.experimental import pallas as pl
from jax.experimental.pallas import tpu as pltpu
from jax.experimental.pallas import tpu_sc as plsc

_B = 256
_P = 40
_N = _B * _P
_E = _N * 16
_H = 64
_WD = 128
_NC = 2
_NS = 16
_NW = _NC * _NS
_EW = _E // _NW
_CH = 128
_NCHUNK = _EW // _CH
_NROWS_SUB = _N // _NS
_EH = _E // 2
_EWH = _EH // _NW
_NCH = _EWH // _CH


def _silu(x):
    return x * jax.nn.sigmoid(x)


@functools.cache
def _sc_mesh():
    return plsc.VectorSubcoreMesh(
        core_axis_name="c", subcore_axis_name="s",
        num_cores=_NC, num_subcores=_NS)


_NBUF = 3


def _gather_kernel_body(tsrc, tdst, src3, dst3, g,
                        idx_s, idx_d, gs, gd, sem_s, sem_d, sem_w):
    c = lax.axis_index("c")
    s = lax.axis_index("s")
    wid = s * _NC + c
    base = wid * _EWH
    pltpu.sync_copy(src3.at[wid], idx_s)
    pltpu.sync_copy(dst3.at[wid], idx_d)

    pend_g = [None] * _NBUF
    pend_w = [None] * _NBUF

    def launch(i):
        b = i % _NBUF
        if pend_w[b] is not None:
            pend_w[b].wait()
        pend_g[b] = (
            pltpu.async_copy(tsrc.at[idx_s.at[i]], gs.at[b], sem_s.at[b]),
            pltpu.async_copy(tdst.at[idx_d.at[i]], gd.at[b], sem_d.at[b]),
        )

    def combine(b, i):
        @pl.loop(0, _CH)
        def _row(r):
            for j in range(4):
                sl = pl.ds(j * 16, 16)
                gs[b, r, sl] = gs[b, r, sl] + gd[b, r, sl]
            sl = pl.ds(64, 16)
            gs[b, r, sl] = gs[b, r, sl] - gd[b, r, sl]

    launch(0)
    launch(1)
    for i in range(_NCH):
        b = i % _NBUF
        d1, d2 = pend_g[b]
        d1.wait()
        d2.wait()
        combine(b, i)
        pend_w[b] = pltpu.async_copy(
            gs.at[b], g.at[pl.ds(base + i * _CH, _CH)], sem_w.at[b])
        if i + 2 < _NCH:
            launch(i + 2)
    for b in range(_NBUF):
        if pend_w[b] is not None:
            pend_w[b].wait()


@functools.cache
def _sc_gather_call():
    return pl.kernel(
        _gather_kernel_body,
        out_type=jax.ShapeDtypeStruct((_EH, _WD), jnp.float32),
        mesh=_sc_mesh(),
        scratch_types=[
            pltpu.VMEM((_NCH, _CH), jnp.int32),
            pltpu.VMEM((_NCH, _CH), jnp.int32),
            pltpu.VMEM((_NBUF, _CH, _WD), jnp.float32),
            pltpu.VMEM((_NBUF, _CH, _WD), jnp.float32),
            pltpu.SemaphoreType.DMA((_NBUF,)),
            pltpu.SemaphoreType.DMA((_NBUF,)),
            pltpu.SemaphoreType.DMA((_NBUF,)),
        ],
        compiler_params=pltpu.CompilerParams(use_tc_tiling_on_sc=True),
    )


def _sc_gather(ts, td, src3, dst3):
    return _sc_gather_call()(ts, td, src3, dst3)


def _scatter_kernel_body(msga, msgb, dsta, dstb, out, idxa, idxb, mv,
                         acc_sh, sem_r):
    c = lax.axis_index("c")
    s = lax.axis_index("s")
    wid = s * _NC + c
    base = wid * _EWH

    @pl.loop(0, _CH)
    def _zrow(r):
        for j in range(_WD // 16):
            mv[0, r, pl.ds(j * 16, 16)] = jnp.zeros((16,), jnp.float32)

    for k in range(_NROWS_SUB // _CH):
        pltpu.sync_copy(mv.at[0],
                        acc_sh.at[pl.ds(s * _NROWS_SUB + k * _CH, _CH)])
    pltpu.sync_copy(dsta.at[wid], idxa)
    pltpu.sync_copy(dstb.at[wid], idxb)
    plsc.subcore_barrier()

    pend = [None, None]

    def piece(i):
        if i < _NCH:
            return msga, idxa, i
        return msgb, idxb, i - _NCH

    def launch(i):
        b = i % 2
        msg, _, j = piece(i)
        pend[b] = pltpu.async_copy(
            msg.at[pl.ds(base + j * _CH, _CH)], mv.at[b], sem_r.at[b])

    launch(0)
    launch(1)
    for i in range(2 * _NCH):
        b = i % 2
        pend[b].wait()
        _, idx, j = piece(i)
        pltpu.sync_copy(mv.at[b], acc_sh.at[idx.at[j]], add=True)
        if i + 2 < 2 * _NCH:
            launch(i + 2)

    plsc.subcore_barrier()
    for k in range(_NROWS_SUB // _CH):
        r0 = s * _NROWS_SUB + k * _CH
        pltpu.sync_copy(acc_sh.at[pl.ds(r0, _CH)], mv.at[0])
        pltpu.sync_copy(mv.at[0], out.at[pl.ds(c * _N + r0, _CH)])


@functools.cache
def _sc_scatter_call():
    return pl.kernel(
        _scatter_kernel_body,
        out_type=jax.ShapeDtypeStruct((_NC * _N, _WD), jnp.float32),
        mesh=_sc_mesh(),
        scratch_types=[
            pltpu.VMEM((_NCH, _CH), jnp.int32),
            pltpu.VMEM((_NCH, _CH), jnp.int32),
            pltpu.VMEM((2, _CH, _WD), jnp.float32),
            pltpu.VMEM_SHARED((_N, _WD), jnp.float32),
            pltpu.SemaphoreType.DMA((2,)),
        ],
        compiler_params=pltpu.CompilerParams(use_tc_tiling_on_sc=True),
    )


def _sc_scatter(msga, msgb, dsta, dstb):
    return _sc_scatter_call()(msga, msgb, dsta, dstb)


_BE = 2048
_BN = 2048


def _rep(shape):
    return pl.BlockSpec(shape, lambda i: tuple(0 for _ in shape))


def _edge_body(g, ea2, sel, lm, wr, wet, w2, b2, wc1, bc1, wc2, out):
    s1 = g[:, :_H]
    xd = g[:, _H:_H + 3]
    ea_bc = jnp.dot(sel[...], ea2[...]) * lm[...]
    ea_we = jnp.dot(ea_bc, wet[...])
    radial = jnp.sum(xd * xd, axis=1, keepdims=True)
    pre = s1 + radial * wr[...] + ea_we
    t = _silu(pre)
    mh = _silu(jnp.dot(t, w2[...]) + b2[...])
    u = _silu(jnp.dot(mh, wc1[...]) + bc1[...])
    sc = jnp.sum(u * wc2[...], axis=1, keepdims=True)
    mx = sc * (xd / (jnp.sqrt(radial) + 1e-30))
    pad = jnp.zeros((mh.shape[0], _WD - _H - 3), jnp.float32)
    out[...] = jnp.concatenate([mh, mx, pad], axis=1)


def _tc_edge(g, ea2, sel, lm, w):
    return pl.pallas_call(
        _edge_body,
        grid=(_EH // _BE,),
        in_specs=[
            pl.BlockSpec((_BE, _WD), lambda i: (i, 0)),
            pl.BlockSpec((_BE // _CH, _CH), lambda i: (i, 0)),
            _rep((_BE, _BE // _CH)), _rep((_BE, _CH)),
            _rep((1, _H)), _rep((_CH, _H)),
            _rep((_H, _H)), _rep((1, _H)),
            _rep((_H, _H)), _rep((1, _H)), _rep((1, _H)),
        ],
        out_specs=pl.BlockSpec((_BE, _WD), lambda i: (i, 0)),
        out_shape=jax.ShapeDtypeStruct((_EH, _WD), jnp.float32),
    )(g, ea2, sel, lm, w["wr"], w["wet"], w["W2"], w["b2"],
      w["Wc1"], w["bc1"], w["wc2"])


def _prep_body(x, w1s, w1d, b1, h0, coord, ts, td):
    h = x[:, :20]
    c3 = x[:, 20:23]
    z1 = jnp.zeros((h.shape[0], 1), jnp.float32)
    cd = jnp.concatenate([c3, z1], axis=1)
    h0[...] = h
    coord[...] = cd
    zs = jnp.zeros((h.shape[0], _WD - _H - 4), jnp.float32)
    ts[...] = jnp.concatenate([jnp.dot(h, w1s[...]) + b1[...], cd, zs], axis=1)
    td[...] = jnp.concatenate([jnp.dot(h, w1d[...]), cd, zs], axis=1)


def _tc_prep(x, w1s, w1d, b1):
    fin = 20
    return pl.pallas_call(
        _prep_body,
        grid=(_N // _BN,),
        in_specs=[
            pl.BlockSpec((_BN, 23), lambda i: (i, 0)),
            _rep((fin, _H)), _rep((fin, _H)), _rep((1, _H)),
        ],
        out_specs=[
            pl.BlockSpec((_BN, 20), lambda i: (i, 0)),
            pl.BlockSpec((_BN, 4), lambda i: (i, 0)),
            pl.BlockSpec((_BN, _WD), lambda i: (i, 0)),
            pl.BlockSpec((_BN, _WD), lambda i: (i, 0)),
        ],
        out_shape=[
            jax.ShapeDtypeStruct((_N, 20), jnp.float32),
            jax.ShapeDtypeStruct((_N, 4), jnp.float32),
            jax.ShapeDtypeStruct((_N, _WD), jnp.float32),
            jax.ShapeDtypeStruct((_N, _WD), jnp.float32),
        ],
    )(x, w1s, w1d, b1)


def _node_mid_body(acc, h_in, coord, wn1a, wn1b, bn1, wn2, bn2,
                   w1s, w1d, b1, h_out, coord_out, ts, td):
    hn = acc[0, :, :_H] + acc[1, :, :_H]
    xn = acc[0, :, _H:_H + 4] + acc[1, :, _H:_H + 4]
    cnew = coord[...] + xn
    t = _silu(jnp.dot(h_in[...], wn1a[...]) + jnp.dot(hn, wn1b[...]) + bn1[...])
    hnew = jnp.dot(t, wn2[...]) + bn2[...]
    h_out[...] = hnew
    coord_out[...] = cnew
    zs = jnp.zeros((hnew.shape[0], _WD - _H - 4), jnp.float32)
    ts[...] = jnp.concatenate([jnp.dot(hnew, w1s[...]) + b1[...], cnew, zs], axis=1)
    td[...] = jnp.concatenate([jnp.dot(hnew, w1d[...]), cnew, zs], axis=1)


def _node_last_body(acc, h_in, wn1a, wn1b, bn1, wn2, bn2, h_out):
    hn = acc[0, :, :_H] + acc[1, :, :_H]
    t = _silu(jnp.dot(h_in[...], wn1a[...]) + jnp.dot(hn, wn1b[...]) + bn1[...])
    h_out[...] = jnp.dot(t, wn2[...]) + bn2[...]


def _tc_node_mid(acc, h_in, coord, fin, wn, wnext):
    return pl.pallas_call(
        _node_mid_body,
        grid=(_N // _BN,),
        in_specs=[
            pl.BlockSpec((2, _BN, _WD), lambda i: (0, i, 0)),
            pl.BlockSpec((_BN, fin), lambda i: (i, 0)),
            pl.BlockSpec((_BN, 4), lambda i: (i, 0)),
            _rep((fin, _H)), _rep((_H, _H)), _rep((1, _H)),
            _rep((_H, _H)), _rep((1, _H)),
            _rep((_H, _H)), _rep((_H, _H)), _rep((1, _H)),
        ],
        out_specs=[
            pl.BlockSpec((_BN, _H), lambda i: (i, 0)),
            pl.BlockSpec((_BN, 4), lambda i: (i, 0)),
            pl.BlockSpec((_BN, _WD), lambda i: (i, 0)),
            pl.BlockSpec((_BN, _WD), lambda i: (i, 0)),
        ],
        out_shape=[
            jax.ShapeDtypeStruct((_N, _H), jnp.float32),
            jax.ShapeDtypeStruct((_N, 4), jnp.float32),
            jax.ShapeDtypeStruct((_N, _WD), jnp.float32),
            jax.ShapeDtypeStruct((_N, _WD), jnp.float32),
        ],
    )(acc, h_in, coord, wn["Wn1a"], wn["Wn1b"], wn["bn1"], wn["Wn2"], wn["bn2"],
      wnext["W1s"], wnext["W1d"], wnext["b1"])


def _tc_node_last(acc, h_in, fin, wn):
    return pl.pallas_call(
        _node_last_body,
        grid=(_N // _BN,),
        in_specs=[
            pl.BlockSpec((2, _BN, _WD), lambda i: (0, i, 0)),
            pl.BlockSpec((_BN, fin), lambda i: (i, 0)),
            _rep((fin, _H)), _rep((_H, _H)), _rep((1, _H)),
            _rep((_H, _H)), _rep((1, _H)),
        ],
        out_specs=pl.BlockSpec((_BN, _H), lambda i: (i, 0)),
        out_shape=jax.ShapeDtypeStruct((_N, _H), jnp.float32),
    )(acc, h_in, wn["Wn1a"], wn["Wn1b"], wn["bn1"], wn["Wn2"], wn["bn2"])


_GA = 16


def _attn_body(h, wq, bq, wk, bk, wv, bv, out):
    hb = h[...]
    q = jnp.dot(hb, wq[...]) + bq[...]
    k = jnp.dot(hb, wk[...]) + bk[...]
    v = jnp.dot(hb, wv[...]) + bv[...]
    s = lax.dot_general(q, k, (((1,), (1,)), ((), ()))) * (1.0 / 8.0)
    r = lax.broadcasted_iota(jnp.int32, s.shape, 0) // _P
    c = lax.broadcasted_iota(jnp.int32, s.shape, 1) // _P
    s = jnp.where(r == c, s, -1e30)
    m = jnp.max(s, axis=1, keepdims=True)
    p = jnp.exp(s - m)
    p = p / jnp.sum(p, axis=1, keepdims=True)
    ao = jnp.dot(p, v)
    pr = lax.broadcasted_iota(jnp.int32, (_GA, _GA * _P), 0)
    pc = lax.broadcasted_iota(jnp.int32, (_GA, _GA * _P), 1) // _P
    pool = jnp.where(pr == pc, 1.0 / _P, 0.0)
    out[...] = jnp.dot(pool, ao)


def _tc_attn(h, wa):
    return pl.pallas_call(
        _attn_body,
        grid=(_B // _GA,),
        in_specs=[
            pl.BlockSpec((_GA * _P, _H), lambda i: (i, 0)),
            _rep((_H, _H)), _rep((1, _H)),
            _rep((_H, _H)), _rep((1, _H)),
            _rep((_H, _H)), _rep((1, _H)),
        ],
        out_specs=pl.BlockSpec((_GA, _H), lambda i: (i, 0)),
        out_shape=jax.ShapeDtypeStruct((_B, _H), jnp.float32),
    )(h, wa["q"]["W"], wa["q"]["b"][None, :], wa["k"]["W"], wa["k"]["b"][None, :],
      wa["v"]["W"], wa["v"]["b"][None, :])


def _tail_body(xg, seq, pp, eps,
               w1, c1, w21, c21, w22, c22, w3, c3, w4, c4,
               p1, d1, p2, d2, wc, cc, wh, ch, wnp, cnp,
               recon, mu_o, lv_o, fin_o, np_o):
    h1 = jax.nn.relu(jnp.dot(seq[...], w1[...]) + c1[...])
    mu = jnp.dot(h1, w21[...]) + c21[...]
    lv = jnp.dot(h1, w22[...]) + c22[...]
    std = jnp.exp(0.5 * lv)
    z0 = mu + eps[...] * std
    ppv = pp[...]
    pe = jax.nn.relu(jnp.dot(ppv, p1[...]) + d1[...])
    pe = jax.nn.relu(jnp.dot(pe, p2[...]) + d2[...])
    z = jnp.concatenate([z0, pe], axis=1)
    h3 = jax.nn.relu(jnp.dot(z, w3[...]) + c3[...])
    recon[...] = jnp.dot(h3, w4[...]) + c4[...]
    mu_o[...] = mu
    lv_o[...] = lv
    xgv = xg[...]
    comb = jnp.concatenate([xgv, z, xgv, z], axis=1)
    fusion = jax.nn.relu(jnp.dot(comb, wc[...]) + cc[...])
    fin_o[...] = jnp.dot(fusion, wh[...]) + ch[...]
    np_o[...] = jnp.dot(fusion, wnp[...]) + cnp[...]


def _tc_tail(xg, seq, pp, eps, params):
    def wb(p):
        return p["W"], p["b"][None, :]

    w1, c1 = wb(params["vae_fc1"])
    w21, c21 = wb(params["vae_fc21"])
    w22, c22 = wb(params["vae_fc22"])
    w3, c3 = wb(params["vae_fc3"])
    w4, c4 = wb(params["vae_fc4"])
    p1, d1 = wb(params["prop1"])
    p2, d2 = wb(params["prop2"])
    wc, cc = wb(params["clf"])
    wh, ch = wb(params["clf_head"])
    wnp, cnp = wb(params["node_head"])
    args = (xg, seq, pp, eps, w1, c1, w21, c21, w22, c22, w3, c3, w4, c4,
            p1, d1, p2, d2, wc, cc, wh, ch, wnp, cnp)
    return pl.pallas_call(
        _tail_body,
        grid=(1,),
        in_specs=[_rep(a.shape) for a in args],
        out_specs=[
            _rep((_B, 800)), _rep((_B, 32)), _rep((_B, 32)),
            _rep((_B, 1)), _rep((_B, 20)),
        ],
        out_shape=[
            jax.ShapeDtypeStruct((_B, 800), jnp.float32),
            jax.ShapeDtypeStruct((_B, 32), jnp.float32),
            jax.ShapeDtypeStruct((_B, 32), jnp.float32),
            jax.ShapeDtypeStruct((_B, 1), jnp.float32),
            jax.ShapeDtypeStruct((_B, 20), jnp.float32),
        ],
    )(*args)


def _edge_weights(lp, fin):
    w1 = lp["edge1"]["W"]
    return {
        "W1s": w1[:fin],
        "W1d": w1[fin:2 * fin],
        "wr": w1[2 * fin:2 * fin + 1],
        "wet": jnp.broadcast_to(w1[2 * fin + 1:2 * fin + 2], (_CH, _H)),
        "b1": lp["edge1"]["b"][None, :],
        "W2": lp["edge2"]["W"],
        "b2": lp["edge2"]["b"][None, :],
        "Wc1": lp["coord1"]["W"],
        "bc1": lp["coord1"]["b"][None, :],
        "wc2": lp["coord2"]["W"].reshape(1, _H),
    }


def _node_weights(lp, fin):
    wn1 = lp["node1"]["W"]
    return {
        "Wn1a": wn1[:fin],
        "Wn1b": wn1[fin:],
        "bn1": lp["node1"]["b"][None, :],
        "Wn2": lp["node2"]["W"],
        "bn2": lp["node2"]["b"][None, :],
    }


def kernel(graph_node_x, graph_edge_index, graph_edge_attr, sequence_data,
           peptide_property, params):
    src = graph_edge_index[0]
    dst = graph_edge_index[1]
    srcA = src[:_EH].reshape(_NW, _NCH, _CH)
    srcB = src[_EH:].reshape(_NW, _NCH, _CH)
    dstA = dst[:_EH].reshape(_NW, _NCH, _CH)
    dstB = dst[_EH:].reshape(_NW, _NCH, _CH)
    eaA = graph_edge_attr[:_EH].reshape(_EH // _CH, _CH)
    eaB = graph_edge_attr[_EH:].reshape(_EH // _CH, _CH)
    rr = jnp.arange(_BE)[:, None]
    sel = (rr // _CH == jnp.arange(_BE // _CH)[None, :]).astype(jnp.float32)
    lm = (rr % _CH == jnp.arange(_CH)[None, :]).astype(jnp.float32)
    eps = jax.random.normal(jax.random.key(42), (_B, 32), jnp.float32)
    eg = params["egnn"]
    fins = [20, 64, 64, 64, 64, 64]
    ew = [_edge_weights(eg[i], fins[i]) for i in range(6)]
    nw = [_node_weights(eg[i], fins[i]) for i in range(6)]

    w0 = ew[0]
    h, coord, ts, td = _tc_prep(graph_node_x, w0["W1s"], w0["W1d"], w0["b1"])
    for l in range(6):
        gA = _sc_gather(ts, td, srcA, dstA)
        msgA = _tc_edge(gA, eaA, sel, lm, ew[l])
        gB = _sc_gather(ts, td, srcB, dstB)
        msgB = _tc_edge(gB, eaB, sel, lm, ew[l])
        acc = _sc_scatter(msgA, msgB, dstA, dstB).reshape(2, _N, _WD)
        if l < 5:
            h, coord, ts, td = _tc_node_mid(acc, h, coord, fins[l], nw[l], ew[l + 1])
        else:
            h = _tc_node_last(acc, h, fins[l], nw[l])

    x_gat = _tc_attn(h, params["attn"])
    return _tc_tail(x_gat, sequence_data, peptide_property, eps, params)


# speedup vs baseline: 1.0523x; 1.0523x over previous
import functools

import jax
import jax.numpy as jnp
from jax import lax
from jax.experimental import pallas as pl
from jax.experimental.pallas import tpu as pltpu
from jax.experimental.pallas import tpu_sc as plsc

_B = 256
_P = 40
_N = _B * _P
_E = _N * 16
_H = 64
_WD = 128
_NC = 2
_NS = 16
_NW = _NC * _NS
_EW = _E // _NW
_CH = 128
_NCHUNK = _EW // _CH
_NROWS_SUB = _N // _NS
_EH = _E // 2
_EWH = _EH // _NW
_NCH = _EWH // _CH


def _silu(x):
    return x * jax.nn.sigmoid(x)


@functools.cache
def _sc_mesh():
    return plsc.VectorSubcoreMesh(
        core_axis_name="c", subcore_axis_name="s",
        num_cores=_NC, num_subcores=_NS)


_NBUF = 3


def _gather_kernel_body(tsrc, tdst, src3, dst3, g,
                        idx_s, idx_d, gs, gd, sem_s, sem_d, sem_w):
    c = lax.axis_index("c")
    s = lax.axis_index("s")
    wid = s * _NC + c
    base = wid * _EWH
    pltpu.sync_copy(src3.at[wid], idx_s)
    pltpu.sync_copy(dst3.at[wid], idx_d)

    pend_g = [None] * _NBUF
    pend_w = [None] * _NBUF

    def launch(i):
        b = i % _NBUF
        if pend_w[b] is not None:
            pend_w[b].wait()
        pend_g[b] = (
            pltpu.async_copy(tsrc.at[idx_s.at[i]], gs.at[b], sem_s.at[b]),
            pltpu.async_copy(tdst.at[idx_d.at[i]], gd.at[b], sem_d.at[b]),
        )

    def combine(b, i):
        @pl.loop(0, _CH)
        def _row(r):
            for j in range(4):
                sl = pl.ds(j * 16, 16)
                gs[b, r, sl] = gs[b, r, sl] + gd[b, r, sl]
            sl = pl.ds(64, 16)
            gs[b, r, sl] = gs[b, r, sl] - gd[b, r, sl]

    launch(0)
    launch(1)
    for i in range(_NCH):
        b = i % _NBUF
        d1, d2 = pend_g[b]
        d1.wait()
        d2.wait()
        combine(b, i)
        pend_w[b] = pltpu.async_copy(
            gs.at[b], g.at[pl.ds(base + i * _CH, _CH)], sem_w.at[b])
        if i + 2 < _NCH:
            launch(i + 2)
    for b in range(_NBUF):
        if pend_w[b] is not None:
            pend_w[b].wait()


@functools.cache
def _sc_gather_call():
    return pl.kernel(
        _gather_kernel_body,
        out_type=jax.ShapeDtypeStruct((_EH, _WD), jnp.float32),
        mesh=_sc_mesh(),
        scratch_types=[
            pltpu.VMEM((_NCH, _CH), jnp.int32),
            pltpu.VMEM((_NCH, _CH), jnp.int32),
            pltpu.VMEM((_NBUF, _CH, _WD), jnp.float32),
            pltpu.VMEM((_NBUF, _CH, _WD), jnp.float32),
            pltpu.SemaphoreType.DMA((_NBUF,)),
            pltpu.SemaphoreType.DMA((_NBUF,)),
            pltpu.SemaphoreType.DMA((_NBUF,)),
        ],
        compiler_params=pltpu.CompilerParams(use_tc_tiling_on_sc=True),
    )


def _sc_gather(ts, td, src3, dst3):
    return _sc_gather_call()(ts, td, src3, dst3)


def _scatter_kernel_body(msga, msgb, dsta, dstb, out, idxa, idxb, mv,
                         acc_sh, sem_r):
    c = lax.axis_index("c")
    s = lax.axis_index("s")
    wid = s * _NC + c
    base = wid * _EWH

    @pl.loop(0, _CH)
    def _zrow(r):
        for j in range(_WD // 16):
            mv[0, r, pl.ds(j * 16, 16)] = jnp.zeros((16,), jnp.float32)

    for k in range(_NROWS_SUB // _CH):
        pltpu.sync_copy(mv.at[0],
                        acc_sh.at[pl.ds(s * _NROWS_SUB + k * _CH, _CH)])
    pltpu.sync_copy(dsta.at[wid], idxa)
    pltpu.sync_copy(dstb.at[wid], idxb)
    plsc.subcore_barrier()

    pend = [None, None]

    def piece(i):
        if i < _NCH:
            return msga, idxa, i
        return msgb, idxb, i - _NCH

    def launch(i):
        b = i % 2
        msg, _, j = piece(i)
        pend[b] = pltpu.async_copy(
            msg.at[pl.ds(base + j * _CH, _CH)], mv.at[b], sem_r.at[b])

    launch(0)
    launch(1)
    for i in range(2 * _NCH):
        b = i % 2
        pend[b].wait()
        _, idx, j = piece(i)
        pltpu.sync_copy(mv.at[b], acc_sh.at[idx.at[j]], add=True)
        if i + 2 < 2 * _NCH:
            launch(i + 2)

    plsc.subcore_barrier()
    for k in range(_NROWS_SUB // _CH):
        r0 = s * _NROWS_SUB + k * _CH
        pltpu.sync_copy(acc_sh.at[pl.ds(r0, _CH)], mv.at[0])
        pltpu.sync_copy(mv.at[0], out.at[pl.ds(c * _N + r0, _CH)])


@functools.cache
def _sc_scatter_call():
    return pl.kernel(
        _scatter_kernel_body,
        out_type=jax.ShapeDtypeStruct((_NC * _N, _WD), jnp.float32),
        mesh=_sc_mesh(),
        scratch_types=[
            pltpu.VMEM((_NCH, _CH), jnp.int32),
            pltpu.VMEM((_NCH, _CH), jnp.int32),
            pltpu.VMEM((2, _CH, _WD), jnp.float32),
            pltpu.VMEM_SHARED((_N, _WD), jnp.float32),
            pltpu.SemaphoreType.DMA((2,)),
        ],
        compiler_params=pltpu.CompilerParams(use_tc_tiling_on_sc=True),
    )


def _sc_scatter(msga, msgb, dsta, dstb):
    return _sc_scatter_call()(msga, msgb, dsta, dstb)


_BE = 4096
_BN = 2048


def _rep(shape):
    return pl.BlockSpec(shape, lambda i: tuple(0 for _ in shape))


def _edge_body(g, ea2, wr, we, w2, b2, wc1, bc1, wc2, out):
    s1 = g[:, :_H]
    xd = g[:, _H:_H + 3]
    ea = ea2[...]
    radial = jnp.sum(xd * xd, axis=1, keepdims=True)
    pre = s1 + radial * wr[...] + ea * we[...]
    t = _silu(pre)
    mh = _silu(jnp.dot(t, w2[...]) + b2[...])
    u = _silu(jnp.dot(mh, wc1[...]) + bc1[...])
    sc = jnp.sum(u * wc2[...], axis=1, keepdims=True)
    mx = sc * (xd / (jnp.sqrt(radial) + 1e-30))
    pad = jnp.zeros((mh.shape[0], _WD - _H - 3), jnp.float32)
    out[...] = jnp.concatenate([mh, mx, pad], axis=1)


def _tc_edge(g, ea2, w):
    return pl.pallas_call(
        _edge_body,
        grid=(_EH // _BE,),
        in_specs=[
            pl.BlockSpec((_BE, _WD), lambda i: (i, 0)),
            pl.BlockSpec((_BE, 1), lambda i: (i, 0)),
            _rep((1, _H)), _rep((1, _H)),
            _rep((_H, _H)), _rep((1, _H)),
            _rep((_H, _H)), _rep((1, _H)), _rep((1, _H)),
        ],
        out_specs=pl.BlockSpec((_BE, _WD), lambda i: (i, 0)),
        out_shape=jax.ShapeDtypeStruct((_EH, _WD), jnp.float32),
    )(g, ea2, w["wr"], w["we"], w["W2"], w["b2"],
      w["Wc1"], w["bc1"], w["wc2"])


def _prep_body(x, w1s, w1d, b1, h0, coord, ts, td):
    h = x[:, :20]
    c3 = x[:, 20:23]
    z1 = jnp.zeros((h.shape[0], 1), jnp.float32)
    cd = jnp.concatenate([c3, z1], axis=1)
    h0[...] = h
    coord[...] = cd
    zs = jnp.zeros((h.shape[0], _WD - _H - 4), jnp.float32)
    ts[...] = jnp.concatenate([jnp.dot(h, w1s[...]) + b1[...], cd, zs], axis=1)
    td[...] = jnp.concatenate([jnp.dot(h, w1d[...]), cd, zs], axis=1)


def _tc_prep(x, w1s, w1d, b1):
    fin = 20
    return pl.pallas_call(
        _prep_body,
        grid=(_N // _BN,),
        in_specs=[
            pl.BlockSpec((_BN, 23), lambda i: (i, 0)),
            _rep((fin, _H)), _rep((fin, _H)), _rep((1, _H)),
        ],
        out_specs=[
            pl.BlockSpec((_BN, 20), lambda i: (i, 0)),
            pl.BlockSpec((_BN, 4), lambda i: (i, 0)),
            pl.BlockSpec((_BN, _WD), lambda i: (i, 0)),
            pl.BlockSpec((_BN, _WD), lambda i: (i, 0)),
        ],
        out_shape=[
            jax.ShapeDtypeStruct((_N, 20), jnp.float32),
            jax.ShapeDtypeStruct((_N, 4), jnp.float32),
            jax.ShapeDtypeStruct((_N, _WD), jnp.float32),
            jax.ShapeDtypeStruct((_N, _WD), jnp.float32),
        ],
    )(x, w1s, w1d, b1)


def _node_mid_body(acc, h_in, coord, wn1a, wn1b, bn1, wn2, bn2,
                   w1s, w1d, b1, h_out, coord_out, ts, td):
    hn = acc[0, :, :_H] + acc[1, :, :_H]
    xn = acc[0, :, _H:_H + 4] + acc[1, :, _H:_H + 4]
    cnew = coord[...] + xn
    t = _silu(jnp.dot(h_in[...], wn1a[...]) + jnp.dot(hn, wn1b[...]) + bn1[...])
    hnew = jnp.dot(t, wn2[...]) + bn2[...]
    h_out[...] = hnew
    coord_out[...] = cnew
    zs = jnp.zeros((hnew.shape[0], _WD - _H - 4), jnp.float32)
    ts[...] = jnp.concatenate([jnp.dot(hnew, w1s[...]) + b1[...], cnew, zs], axis=1)
    td[...] = jnp.concatenate([jnp.dot(hnew, w1d[...]), cnew, zs], axis=1)


def _node_last_body(acc, h_in, wn1a, wn1b, bn1, wn2, bn2, h_out):
    hn = acc[0, :, :_H] + acc[1, :, :_H]
    t = _silu(jnp.dot(h_in[...], wn1a[...]) + jnp.dot(hn, wn1b[...]) + bn1[...])
    h_out[...] = jnp.dot(t, wn2[...]) + bn2[...]


def _tc_node_mid(acc, h_in, coord, fin, wn, wnext):
    return pl.pallas_call(
        _node_mid_body,
        grid=(_N // _BN,),
        in_specs=[
            pl.BlockSpec((2, _BN, _WD), lambda i: (0, i, 0)),
            pl.BlockSpec((_BN, fin), lambda i: (i, 0)),
            pl.BlockSpec((_BN, 4), lambda i: (i, 0)),
            _rep((fin, _H)), _rep((_H, _H)), _rep((1, _H)),
            _rep((_H, _H)), _rep((1, _H)),
            _rep((_H, _H)), _rep((_H, _H)), _rep((1, _H)),
        ],
        out_specs=[
            pl.BlockSpec((_BN, _H), lambda i: (i, 0)),
            pl.BlockSpec((_BN, 4), lambda i: (i, 0)),
            pl.BlockSpec((_BN, _WD), lambda i: (i, 0)),
            pl.BlockSpec((_BN, _WD), lambda i: (i, 0)),
        ],
        out_shape=[
            jax.ShapeDtypeStruct((_N, _H), jnp.float32),
            jax.ShapeDtypeStruct((_N, 4), jnp.float32),
            jax.ShapeDtypeStruct((_N, _WD), jnp.float32),
            jax.ShapeDtypeStruct((_N, _WD), jnp.float32),
        ],
    )(acc, h_in, coord, wn["Wn1a"], wn["Wn1b"], wn["bn1"], wn["Wn2"], wn["bn2"],
      wnext["W1s"], wnext["W1d"], wnext["b1"])


def _tc_node_last(acc, h_in, fin, wn):
    return pl.pallas_call(
        _node_last_body,
        grid=(_N // _BN,),
        in_specs=[
            pl.BlockSpec((2, _BN, _WD), lambda i: (0, i, 0)),
            pl.BlockSpec((_BN, fin), lambda i: (i, 0)),
            _rep((fin, _H)), _rep((_H, _H)), _rep((1, _H)),
            _rep((_H, _H)), _rep((1, _H)),
        ],
        out_specs=pl.BlockSpec((_BN, _H), lambda i: (i, 0)),
        out_shape=jax.ShapeDtypeStruct((_N, _H), jnp.float32),
    )(acc, h_in, wn["Wn1a"], wn["Wn1b"], wn["bn1"], wn["Wn2"], wn["bn2"])


_GA = 16


def _attn_body(h, wq, bq, wk, bk, wv, bv, out):
    hb = h[...]
    q = jnp.dot(hb, wq[...]) + bq[...]
    k = jnp.dot(hb, wk[...]) + bk[...]
    v = jnp.dot(hb, wv[...]) + bv[...]
    s = lax.dot_general(q, k, (((1,), (1,)), ((), ()))) * (1.0 / 8.0)
    r = lax.broadcasted_iota(jnp.int32, s.shape, 0) // _P
    c = lax.broadcasted_iota(jnp.int32, s.shape, 1) // _P
    s = jnp.where(r == c, s, -1e30)
    m = jnp.max(s, axis=1, keepdims=True)
    p = jnp.exp(s - m)
    p = p / jnp.sum(p, axis=1, keepdims=True)
    ao = jnp.dot(p, v)
    pr = lax.broadcasted_iota(jnp.int32, (_GA, _GA * _P), 0)
    pc = lax.broadcasted_iota(jnp.int32, (_GA, _GA * _P), 1) // _P
    pool = jnp.where(pr == pc, 1.0 / _P, 0.0)
    out[...] = jnp.dot(pool, ao)


def _tc_attn(h, wa):
    return pl.pallas_call(
        _attn_body,
        grid=(_B // _GA,),
        in_specs=[
            pl.BlockSpec((_GA * _P, _H), lambda i: (i, 0)),
            _rep((_H, _H)), _rep((1, _H)),
            _rep((_H, _H)), _rep((1, _H)),
            _rep((_H, _H)), _rep((1, _H)),
        ],
        out_specs=pl.BlockSpec((_GA, _H), lambda i: (i, 0)),
        out_shape=jax.ShapeDtypeStruct((_B, _H), jnp.float32),
    )(h, wa["q"]["W"], wa["q"]["b"][None, :], wa["k"]["W"], wa["k"]["b"][None, :],
      wa["v"]["W"], wa["v"]["b"][None, :])


def _tail_body(xg, seq, pp, eps,
               w1, c1, w21, c21, w22, c22, w3, c3, w4, c4,
               p1, d1, p2, d2, wc, cc, wh, ch, wnp, cnp,
               recon, mu_o, lv_o, fin_o, np_o):
    h1 = jax.nn.relu(jnp.dot(seq[...], w1[...]) + c1[...])
    mu = jnp.dot(h1, w21[...]) + c21[...]
    lv = jnp.dot(h1, w22[...]) + c22[...]
    std = jnp.exp(0.5 * lv)
    z0 = mu + eps[...] * std
    ppv = pp[...]
    pe = jax.nn.relu(jnp.dot(ppv, p1[...]) + d1[...])
    pe = jax.nn.relu(jnp.dot(pe, p2[...]) + d2[...])
    z = jnp.concatenate([z0, pe], axis=1)
    h3 = jax.nn.relu(jnp.dot(z, w3[...]) + c3[...])
    recon[...] = jnp.dot(h3, w4[...]) + c4[...]
    mu_o[...] = mu
    lv_o[...] = lv
    xgv = xg[...]
    comb = jnp.concatenate([xgv, z, xgv, z], axis=1)
    fusion = jax.nn.relu(jnp.dot(comb, wc[...]) + cc[...])
    fin_o[...] = jnp.dot(fusion, wh[...]) + ch[...]
    np_o[...] = jnp.dot(fusion, wnp[...]) + cnp[...]


def _tc_tail(xg, seq, pp, eps, params):
    def wb(p):
        return p["W"], p["b"][None, :]

    w1, c1 = wb(params["vae_fc1"])
    w21, c21 = wb(params["vae_fc21"])
    w22, c22 = wb(params["vae_fc22"])
    w3, c3 = wb(params["vae_fc3"])
    w4, c4 = wb(params["vae_fc4"])
    p1, d1 = wb(params["prop1"])
    p2, d2 = wb(params["prop2"])
    wc, cc = wb(params["clf"])
    wh, ch = wb(params["clf_head"])
    wnp, cnp = wb(params["node_head"])
    args = (xg, seq, pp, eps, w1, c1, w21, c21, w22, c22, w3, c3, w4, c4,
            p1, d1, p2, d2, wc, cc, wh, ch, wnp, cnp)
    return pl.pallas_call(
        _tail_body,
        grid=(1,),
        in_specs=[_rep(a.shape) for a in args],
        out_specs=[
            _rep((_B, 800)), _rep((_B, 32)), _rep((_B, 32)),
            _rep((_B, 1)), _rep((_B, 20)),
        ],
        out_shape=[
            jax.ShapeDtypeStruct((_B, 800), jnp.float32),
            jax.ShapeDtypeStruct((_B, 32), jnp.float32),
            jax.ShapeDtypeStruct((_B, 32), jnp.float32),
            jax.ShapeDtypeStruct((_B, 1), jnp.float32),
            jax.ShapeDtypeStruct((_B, 20), jnp.float32),
        ],
    )(*args)


def _edge_weights(lp, fin):
    w1 = lp["edge1"]["W"]
    return {
        "W1s": w1[:fin],
        "W1d": w1[fin:2 * fin],
        "wr": w1[2 * fin:2 * fin + 1],
        "we": w1[2 * fin + 1:2 * fin + 2],
        "b1": lp["edge1"]["b"][None, :],
        "W2": lp["edge2"]["W"],
        "b2": lp["edge2"]["b"][None, :],
        "Wc1": lp["coord1"]["W"],
        "bc1": lp["coord1"]["b"][None, :],
        "wc2": lp["coord2"]["W"].reshape(1, _H),
    }


def _node_weights(lp, fin):
    wn1 = lp["node1"]["W"]
    return {
        "Wn1a": wn1[:fin],
        "Wn1b": wn1[fin:],
        "bn1": lp["node1"]["b"][None, :],
        "Wn2": lp["node2"]["W"],
        "bn2": lp["node2"]["b"][None, :],
    }


def kernel(graph_node_x, graph_edge_index, graph_edge_attr, sequence_data,
           peptide_property, params):
    src = graph_edge_index[0]
    dst = graph_edge_index[1]
    srcA = src[:_EH].reshape(_NW, _NCH, _CH)
    srcB = src[_EH:].reshape(_NW, _NCH, _CH)
    dstA = dst[:_EH].reshape(_NW, _NCH, _CH)
    dstB = dst[_EH:].reshape(_NW, _NCH, _CH)
    eaA = graph_edge_attr[:_EH]
    eaB = graph_edge_attr[_EH:]
    eps = jax.random.normal(jax.random.key(42), (_B, 32), jnp.float32)
    eg = params["egnn"]
    fins = [20, 64, 64, 64, 64, 64]
    ew = [_edge_weights(eg[i], fins[i]) for i in range(6)]
    nw = [_node_weights(eg[i], fins[i]) for i in range(6)]

    w0 = ew[0]
    h, coord, ts, td = _tc_prep(graph_node_x, w0["W1s"], w0["W1d"], w0["b1"])
    for l in range(6):
        gA = _sc_gather(ts, td, srcA, dstA)
        msgA = _tc_edge(gA, eaA, ew[l])
        gB = _sc_gather(ts, td, srcB, dstB)
        msgB = _tc_edge(gB, eaB, ew[l])
        acc = _sc_scatter(msgA, msgB, dstA, dstB).reshape(2, _N, _WD)
        if l < 5:
            h, coord, ts, td = _tc_node_mid(acc, h, coord, fins[l], nw[l], ew[l + 1])
        else:
            h = _tc_node_last(acc, h, fins[l], nw[l])

    x_gat = _tc_attn(h, params["attn"])
    return _tc_tail(x_gat, sequence_data, peptide_property, eps, params)
